# baseline (speedup 1.0000x reference)
import jax
import jax.numpy as jnp
from jax.experimental import pallas as pl


def _bcast_body(w_ref, o_ref):
    o_ref[...] = w_ref[...][None, :, :]


def kernel(x, W):
    B, S = x.shape
    D = W.shape[1]
    SBLK = 2048
    assert S % SBLK == 0
    out = pl.pallas_call(
        _bcast_body,
        grid=(S // SBLK, B),
        in_specs=[pl.BlockSpec((SBLK, D), lambda s, b: (s, 0))],
        out_specs=pl.BlockSpec((1, SBLK, D), lambda s, b: (b, s, 0)),
        out_shape=jax.ShapeDtypeStruct((B, S, D), W.dtype),
    )(W[:S])
    return out


# speedup vs baseline: 1.1293x; 1.1293x over previous
import jax
import jax.numpy as jnp
from jax.experimental import pallas as pl


def _bcast_body(w_ref, o_ref):
    o_ref[...] = jnp.broadcast_to(w_ref[...][None, :, :], o_ref.shape)


def kernel(x, W):
    B, S = x.shape
    D = W.shape[1]
    SBLK = 1024
    assert S % SBLK == 0
    out = pl.pallas_call(
        _bcast_body,
        grid=(S // SBLK,),
        in_specs=[pl.BlockSpec((SBLK, D), lambda s: (s, 0))],
        out_specs=pl.BlockSpec((B, SBLK, D), lambda s: (0, s, 0)),
        out_shape=jax.ShapeDtypeStruct((B, S, D), W.dtype),
    )(W[:S])
    return out
